# baseline (speedup 1.0000x reference)
import jax
import jax.numpy as jnp
from jax import lax
from jax.experimental import pallas as pl
from jax.experimental.pallas import tpu as pltpu
from jax.experimental.pallas import tpu_sc as plsc

TOKENS = 4096
DMODEL = 2048
SIZE = 8
LANES = 16

NUM_CORES = 2
NUM_SUBCORES = 16
NUM_WORKERS = NUM_CORES * NUM_SUBCORES
TOK_PER_WORKER = TOKENS // NUM_WORKERS
CHUNK_TOKENS = 2
NUM_CHUNKS = TOK_PER_WORKER // CHUNK_TOKENS
NBUF = 2
GATHER_ITERS = DMODEL // LANES


def _select_body(in_hbm, sel_hbm, out_hbm, sel_v, bufs, obufs, in_sems, out_sems):
    cid = lax.axis_index("c")
    sid = lax.axis_index("s")
    wid = sid * NUM_CORES + cid
    t_base = wid * TOK_PER_WORKER

    pltpu.sync_copy(sel_hbm, sel_v)
    selv = sel_v[...]
    iota = lax.iota(jnp.int32, LANES)

    def in_copy(i, b):
        t0 = t_base + i * CHUNK_TOKENS
        return pltpu.make_async_copy(
            in_hbm.at[pl.ds(t0, CHUNK_TOKENS)], bufs.at[b], in_sems.at[b]
        )

    def out_copy(i, b):
        t0 = t_base + i * CHUNK_TOKENS
        return pltpu.make_async_copy(
            obufs.at[b], out_hbm.at[pl.ds(t0, CHUNK_TOKENS)], out_sems.at[b]
        )

    def gather_chunk(b):
        for t in range(CHUNK_TOKENS):
            src = bufs.at[b, t]

            def step(j, _):
                d_idx = j * LANES + iota
                v = plsc.load_gather(src, [d_idx, selv])
                obufs[b, t, pl.ds(j * LANES, LANES)] = v
                return 0

            lax.fori_loop(0, GATHER_ITERS, step, 0, unroll=4)

    for b in range(NBUF):
        in_copy(b, b).start()

    for b in range(NBUF):
        in_copy(b, b).wait()
        gather_chunk(b)
        out_copy(b, b).start()
        in_copy(b + NBUF, b).start()

    def pair_body(p, _):
        for b in range(NBUF):
            i = p * NBUF + b
            in_copy(i, b).wait()
            out_copy(i - NBUF, b).wait()
            gather_chunk(b)
            out_copy(i, b).start()
            in_copy(i + NBUF, b).start()
        return 0

    lax.fori_loop(1, NUM_CHUNKS // NBUF - 1, pair_body, 0)

    for b in range(NBUF):
        i = NUM_CHUNKS - NBUF + b
        in_copy(i, b).wait()
        out_copy(i - NBUF, b).wait()
        gather_chunk(b)
        out_copy(i, b).start()
    for b in range(NBUF):
        out_copy(NUM_CHUNKS - NBUF + b, b).wait()


@jax.jit
def _sc_select(input, sel16):
    mesh = plsc.VectorSubcoreMesh(core_axis_name="c", subcore_axis_name="s")
    return pl.kernel(
        _select_body,
        out_type=jax.ShapeDtypeStruct((TOKENS, DMODEL), jnp.float32),
        mesh=mesh,
        scratch_types=[
            pltpu.VMEM((16,), jnp.int32),
            pltpu.VMEM((NBUF, CHUNK_TOKENS, DMODEL, SIZE), jnp.float32),
            pltpu.VMEM((NBUF, CHUNK_TOKENS, DMODEL), jnp.float32),
            pltpu.SemaphoreType.DMA((NBUF,)),
            pltpu.SemaphoreType.DMA((NBUF,)),
        ],
        compiler_params=pltpu.CompilerParams(
            use_tc_tiling_on_sc=False, needs_layout_passes=False
        ),
    )(input, sel16)


def kernel(input, total_loss, pre_sels, weight):
    del total_loss, weight
    sel16 = jnp.broadcast_to(pre_sels.astype(jnp.int32), (16,))
    return _sc_select(input, sel16)


# speedup vs baseline: 6.1307x; 6.1307x over previous
import jax
import jax.numpy as jnp
from jax import lax
from jax.experimental import pallas as pl
from jax.experimental.pallas import tpu as pltpu
from jax.experimental.pallas import tpu_sc as plsc

TOKENS = 4096
DMODEL = 2048
SIZE = 8
LANES = 16

NUM_CORES = 2
NUM_SUBCORES = 16
NUM_WORKERS = NUM_CORES * NUM_SUBCORES
TOK_PER_WORKER = TOKENS // NUM_WORKERS

CHUNK_TOKENS = 8
CHUNK_COLS = 4096
CHUNK_D = CHUNK_COLS // SIZE
COL_CHUNKS = (DMODEL * SIZE) // CHUNK_COLS
TOK_GROUPS = TOK_PER_WORKER // CHUNK_TOKENS
NUM_CHUNKS = TOK_GROUPS * COL_CHUNKS
NBUF = 2
GVECS = CHUNK_D // LANES


def _select_body(in_hbm, sel_hbm, out_hbm, sel_v, bufs, obufs, in_sems, out_sems):
    cid = lax.axis_index("c")
    sid = lax.axis_index("s")
    wid = sid * NUM_CORES + cid
    t_base = wid * TOK_PER_WORKER

    pltpu.sync_copy(sel_hbm, sel_v)
    selv = sel_v[...]
    base_cols = selv + 8 * lax.iota(jnp.int32, LANES)

    def chunk_coords(i):
        tg = i // COL_CHUNKS
        cc = i % COL_CHUNKS
        t0 = pl.multiple_of(t_base + tg * CHUNK_TOKENS, CHUNK_TOKENS)
        m0 = pl.multiple_of(cc * CHUNK_COLS, CHUNK_COLS)
        return t0, m0

    def in_copy(i, b):
        t0, m0 = chunk_coords(i)
        return pltpu.make_async_copy(
            in_hbm.at[pl.ds(t0, CHUNK_TOKENS), pl.ds(m0, CHUNK_COLS)],
            bufs.at[b],
            in_sems.at[b],
        )

    def out_copy(i, b):
        t0, m0 = chunk_coords(i)
        return pltpu.make_async_copy(
            obufs.at[b],
            out_hbm.at[pl.ds(t0, CHUNK_TOKENS), pl.ds(pl.multiple_of(m0 // SIZE, CHUNK_D), CHUNK_D)],
            out_sems.at[b],
        )

    def gather_chunk(b):
        def row(t, _):
            tv = jnp.full((LANES,), t, jnp.int32)
            for g in range(GVECS):
                v = plsc.load_gather(bufs.at[b], [tv, base_cols + g * 128])
                obufs[b, t, pl.ds(g * LANES, LANES)] = v
            return 0

        lax.fori_loop(0, CHUNK_TOKENS, row, 0)

    for b in range(NBUF):
        in_copy(b, b).start()

    for b in range(NBUF):
        in_copy(b, b).wait()
        gather_chunk(b)
        out_copy(b, b).start()
        in_copy(b + NBUF, b).start()

    def pair_body(p, _):
        for b in range(NBUF):
            i = p * NBUF + b
            in_copy(i, b).wait()
            out_copy(i - NBUF, b).wait()
            gather_chunk(b)
            out_copy(i, b).start()
            in_copy(i + NBUF, b).start()
        return 0

    lax.fori_loop(1, NUM_CHUNKS // NBUF - 1, pair_body, 0)

    for b in range(NBUF):
        i = NUM_CHUNKS - NBUF + b
        in_copy(i, b).wait()
        out_copy(i - NBUF, b).wait()
        gather_chunk(b)
        out_copy(i, b).start()
    for b in range(NBUF):
        out_copy(NUM_CHUNKS - NBUF + b, b).wait()


@jax.jit
def _sc_select(input2d, sel16):
    mesh = plsc.VectorSubcoreMesh(core_axis_name="c", subcore_axis_name="s")
    return pl.kernel(
        _select_body,
        out_type=jax.ShapeDtypeStruct((TOKENS, DMODEL), jnp.float32),
        mesh=mesh,
        scratch_types=[
            pltpu.VMEM((16,), jnp.int32),
            pltpu.VMEM((NBUF, CHUNK_TOKENS, CHUNK_COLS), jnp.float32),
            pltpu.VMEM((NBUF, CHUNK_TOKENS, CHUNK_D), jnp.float32),
            pltpu.SemaphoreType.DMA((NBUF,)),
            pltpu.SemaphoreType.DMA((NBUF,)),
        ],
        compiler_params=pltpu.CompilerParams(needs_layout_passes=False),
    )(input2d, sel16)


def kernel(input, total_loss, pre_sels, weight):
    del total_loss, weight
    input2d = input.reshape(TOKENS, DMODEL * SIZE)
    sel16 = jnp.broadcast_to(pre_sels.astype(jnp.int32), (16,))
    return _sc_select(input2d, sel16)
